# baseline (speedup 1.0000x reference)
import functools

import jax
import jax.numpy as jnp
from jax import lax
from jax.experimental import pallas as pl
from jax.experimental.pallas import tpu as pltpu
from jax.experimental.pallas import tpu_sc as plsc

D = 64
HIST = 50
NC = 2
NS = 16
NW = NC * NS
S = 128
G = 4


def _pool_body(idx_hbm, table_hbm, out_hbm, idx_v, acc_v, sem0, sem1):
  wid = lax.axis_index("s") * NC + lax.axis_index("c")
  g0 = wid * G
  pltpu.sync_copy(idx_hbm.at[:, pl.ds(g0, G), :], idx_v)
  first = [
      pltpu.async_copy(table_hbm.at[idx_v.at[0, g]], acc_v.at[g], sem0)
      for g in range(G)
  ]
  for cp in first:
    cp.wait()

  def issue(j, carry):
    for g in range(G):
      pltpu.async_copy(
          table_hbm.at[idx_v.at[j, g]], acc_v.at[g], sem1, add=True
      )
    return carry

  lax.fori_loop(1, HIST, issue, 0)

  def drain(i, carry):
    pltpu.make_async_copy(
        table_hbm.at[idx_v.at[0, 0]], acc_v.at[0], sem1
    ).wait()
    return carry

  lax.fori_loop(0, (HIST - 1) * G, drain, 0)

  pltpu.sync_copy(acc_v, out_hbm.at[pl.ds(g0, G)])


_pool_call = pl.kernel(
    _pool_body,
    out_type=jax.ShapeDtypeStruct((NW * G, S, D), jnp.float32),
    mesh=plsc.VectorSubcoreMesh(core_axis_name="c", subcore_axis_name="s"),
    scratch_types=[
        pltpu.VMEM((HIST, G, S), jnp.int32),
        pltpu.VMEM((G, S, D), jnp.float32),
        pltpu.SemaphoreType.DMA,
        pltpu.SemaphoreType.DMA,
    ],
    compiler_params=pltpu.CompilerParams(use_tc_tiling_on_sc=False),
)


_V = 1000000
_W = 16384
_H = _W // 2
_NB = (_V + _W - 1) // _W


def _detile_body(a_ref, o_ref):
  x = a_ref[...]
  o_ref[...] = jnp.concatenate([x[:, :_H], x[:, _H:]], axis=0).T


def _detile(table_t):
  return pl.pallas_call(
      _detile_body,
      grid=(_NB,),
      in_specs=[pl.BlockSpec((64, _W), lambda i: (0, i))],
      out_specs=pl.BlockSpec((_H, 128), lambda i: (i, 0)),
      out_shape=jax.ShapeDtypeStruct((_NB * _H, 128), jnp.float32),
  )(table_t)


def _mlp_body(x_ref, w1t_ref, b1_ref, w2_ref, b2_ref, o_ref):
  x = x_ref[...] * (1.0 / HIST)
  h = jnp.dot(x, w1t_ref[...], preferred_element_type=jnp.float32)
  h = jnp.maximum(h + b1_ref[...], 0.0)
  z = jnp.sum(h * w2_ref[...], axis=1, keepdims=True) + b2_ref[0, 0]
  o_ref[...] = 1.0 / (1.0 + jnp.exp(-z))


@jax.jit
def kernel(tag_idxs, table, W1, b1, W2, b2):
  batch = tag_idxs.shape[0]
  lin = _detile(table.T).reshape(2 * _NB * _H, D)
  t = tag_idxs.astype(jnp.int32)
  r = t & (_W - 1)
  t = (t - r) + ((r & (_H - 1)) << 1) + (r >> (_H.bit_length() - 1))
  idx3 = t.T.reshape(HIST, NW * G, S)
  pooled = _pool_call(idx3, lin).reshape(batch, D)
  mb = 1024
  out = pl.pallas_call(
      _mlp_body,
      grid=(batch // mb,),
      in_specs=[
          pl.BlockSpec((mb, D), lambda i: (i, 0)),
          pl.BlockSpec((D, D), lambda i: (0, 0)),
          pl.BlockSpec((1, D), lambda i: (0, 0)),
          pl.BlockSpec((1, D), lambda i: (0, 0)),
          pl.BlockSpec((1, 1), lambda i: (0, 0)),
      ],
      out_specs=pl.BlockSpec((mb, 1), lambda i: (i, 0)),
      out_shape=jax.ShapeDtypeStruct((batch, 1), jnp.float32),
  )(pooled, W1.T, b1.reshape(1, D), W2, b2.reshape(1, 1))
  return out


# speedup vs baseline: 1.0196x; 1.0196x over previous
import functools

import jax
import jax.numpy as jnp
from jax import lax
from jax.experimental import pallas as pl
from jax.experimental.pallas import tpu as pltpu
from jax.experimental.pallas import tpu_sc as plsc

D = 64
HIST = 50
NC = 2
NS = 16
NW = NC * NS
S = 64
G = 8


def _pool_body(idx_hbm, table_hbm, out_hbm, idx_v, acc_v, sem0, sem1):
  wid = lax.axis_index("s") * NC + lax.axis_index("c")
  g0 = wid * G
  pltpu.sync_copy(idx_hbm.at[:, pl.ds(g0, G), :], idx_v)
  first = [
      pltpu.async_copy(table_hbm.at[idx_v.at[0, g]], acc_v.at[g], sem0)
      for g in range(G)
  ]
  for cp in first:
    cp.wait()

  def issue(j, carry):
    for g in range(G):
      pltpu.async_copy(
          table_hbm.at[idx_v.at[j, g]], acc_v.at[g], sem1, add=True
      )
    return carry

  lax.fori_loop(1, HIST, issue, 0)

  def drain(i, carry):
    pltpu.make_async_copy(
        table_hbm.at[idx_v.at[0, 0]], acc_v.at[0], sem1
    ).wait()
    return carry

  lax.fori_loop(0, (HIST - 1) * G, drain, 0)

  pltpu.sync_copy(acc_v, out_hbm.at[pl.ds(g0, G)])


_pool_call = pl.kernel(
    _pool_body,
    out_type=jax.ShapeDtypeStruct((NW * G, S, D), jnp.float32),
    mesh=plsc.VectorSubcoreMesh(core_axis_name="c", subcore_axis_name="s"),
    scratch_types=[
        pltpu.VMEM((HIST, G, S), jnp.int32),
        pltpu.VMEM((G, S, D), jnp.float32),
        pltpu.SemaphoreType.DMA,
        pltpu.SemaphoreType.DMA,
    ],
    compiler_params=pltpu.CompilerParams(use_tc_tiling_on_sc=False),
)


_V = 1000000
_W = 16384
_H = _W // 2
_NB = (_V + _W - 1) // _W


def _detile_body(a_ref, o_ref):
  x = a_ref[...]
  o_ref[...] = jnp.concatenate([x[:, :_H], x[:, _H:]], axis=0).T


def _detile(table_t):
  return pl.pallas_call(
      _detile_body,
      grid=(_NB,),
      in_specs=[pl.BlockSpec((64, _W), lambda i: (0, i))],
      out_specs=pl.BlockSpec((_H, 128), lambda i: (i, 0)),
      out_shape=jax.ShapeDtypeStruct((_NB * _H, 128), jnp.float32),
  )(table_t)


def _mlp_body(x_ref, w1t_ref, b1_ref, w2_ref, b2_ref, o_ref):
  x = x_ref[...] * (1.0 / HIST)
  h = jnp.dot(x, w1t_ref[...], preferred_element_type=jnp.float32)
  h = jnp.maximum(h + b1_ref[...], 0.0)
  z = jnp.sum(h * w2_ref[...], axis=1, keepdims=True) + b2_ref[0, 0]
  o_ref[...] = 1.0 / (1.0 + jnp.exp(-z))


@jax.jit
def kernel(tag_idxs, table, W1, b1, W2, b2):
  batch = tag_idxs.shape[0]
  lin = _detile(table.T).reshape(2 * _NB * _H, D)
  t = tag_idxs.astype(jnp.int32)
  r = t & (_W - 1)
  t = (t - r) + ((r & (_H - 1)) << 1) + (r >> (_H.bit_length() - 1))
  idx3 = t.T.reshape(HIST, NW * G, S)
  pooled = _pool_call(idx3, lin).reshape(batch, D)
  out = pl.pallas_call(
      _mlp_body,
      out_shape=jax.ShapeDtypeStruct((batch, 1), jnp.float32),
  )(pooled, W1.T, b1.reshape(1, D), W2, b2.reshape(1, 1))
  return out


# speedup vs baseline: 1.0403x; 1.0203x over previous
import functools

import jax
import jax.numpy as jnp
from jax import lax
from jax.experimental import pallas as pl
from jax.experimental.pallas import tpu as pltpu
from jax.experimental.pallas import tpu_sc as plsc

D = 64
HIST = 50
NC = 2
NS = 16
NW = NC * NS
S = 128
G = 4


def _pool_body(idx_hbm, table_hbm, out_hbm, idx_v, acc_v, sem0, sem1):
  wid = lax.axis_index("s") * NC + lax.axis_index("c")
  g0 = wid * G
  pltpu.sync_copy(idx_hbm.at[:, pl.ds(g0, G), :], idx_v)
  first = [
      pltpu.async_copy(table_hbm.at[idx_v.at[0, g]], acc_v.at[g], sem0)
      for g in range(G)
  ]
  for cp in first:
    cp.wait()

  def issue(j, carry):
    for g in range(G):
      pltpu.async_copy(
          table_hbm.at[idx_v.at[j, g]], acc_v.at[g], sem1, add=True
      )
    return carry

  lax.fori_loop(1, HIST, issue, 0)

  def drain(i, carry):
    pltpu.make_async_copy(
        table_hbm.at[idx_v.at[0, 0]], acc_v.at[0], sem1
    ).wait()
    return carry

  lax.fori_loop(0, (HIST - 1) * G, drain, 0)

  pltpu.sync_copy(acc_v, out_hbm.at[pl.ds(g0, G)])


_pool_call = pl.kernel(
    _pool_body,
    out_type=jax.ShapeDtypeStruct((NW * G, S, D), jnp.float32),
    mesh=plsc.VectorSubcoreMesh(core_axis_name="c", subcore_axis_name="s"),
    scratch_types=[
        pltpu.VMEM((HIST, G, S), jnp.int32),
        pltpu.VMEM((G, S, D), jnp.float32),
        pltpu.SemaphoreType.DMA,
        pltpu.SemaphoreType.DMA,
    ],
    compiler_params=pltpu.CompilerParams(use_tc_tiling_on_sc=False),
)


_V = 1000000
_W = 32768
_H = _W // 2
_NB = (_V + _W - 1) // _W


def _detile_body(a_ref, o_ref):
  x = a_ref[...]
  o_ref[...] = jnp.concatenate([x[:, :_H], x[:, _H:]], axis=0).T


def _detile(table_t):
  return pl.pallas_call(
      _detile_body,
      grid=(_NB,),
      in_specs=[pl.BlockSpec((64, _W), lambda i: (0, i))],
      out_specs=pl.BlockSpec((_H, 128), lambda i: (i, 0)),
      out_shape=jax.ShapeDtypeStruct((_NB * _H, 128), jnp.float32),
  )(table_t)


def _mlp_body(x_ref, w1t_ref, b1_ref, w2_ref, b2_ref, o_ref):
  x = x_ref[...] * (1.0 / HIST)
  h = jnp.dot(x, w1t_ref[...], preferred_element_type=jnp.float32)
  h = jnp.maximum(h + b1_ref[...], 0.0)
  z = jnp.sum(h * w2_ref[...], axis=1, keepdims=True) + b2_ref[0, 0]
  o_ref[...] = 1.0 / (1.0 + jnp.exp(-z))


@jax.jit
def kernel(tag_idxs, table, W1, b1, W2, b2):
  batch = tag_idxs.shape[0]
  lin = _detile(table.T).reshape(2 * _NB * _H, D)
  t = tag_idxs.astype(jnp.int32)
  r = t & (_W - 1)
  t = (t - r) + ((r & (_H - 1)) << 1) + (r >> (_H.bit_length() - 1))
  idx3 = t.T.reshape(HIST, NW * G, S)
  pooled = _pool_call(idx3, lin).reshape(batch, D)
  out = pl.pallas_call(
      _mlp_body,
      out_shape=jax.ShapeDtypeStruct((batch, 1), jnp.float32),
  )(pooled, W1.T, b1.reshape(1, D), W2, b2.reshape(1, 1))
  return out


# speedup vs baseline: 1.0413x; 1.0009x over previous
import jax
import jax.numpy as jnp
from jax import lax
from jax.experimental import pallas as pl
from jax.experimental.pallas import tpu as pltpu
from jax.experimental.pallas import tpu_sc as plsc

D = 64
HIST = 50
NC = 2
NS = 16
NW = NC * NS
S = 128
G = 4


def _pool_body(idx_hbm, table_hbm, out_hbm, idx_v, acc_v, sem0, sem1):
  wid = lax.axis_index("s") * NC + lax.axis_index("c")
  g0 = wid * G
  pltpu.sync_copy(idx_hbm.at[:, pl.ds(g0, G), :], idx_v)
  first = [
      pltpu.async_copy(table_hbm.at[idx_v.at[0, g]], acc_v.at[g], sem0)
      for g in range(G)
  ]
  for cp in first:
    cp.wait()

  def issue(j, carry):
    for g in range(G):
      pltpu.async_copy(
          table_hbm.at[idx_v.at[j, g]], acc_v.at[g], sem1, add=True
      )
    return carry

  lax.fori_loop(1, HIST, issue, 0)

  def drain(i, carry):
    pltpu.make_async_copy(
        table_hbm.at[idx_v.at[0, 0]], acc_v.at[0], sem1
    ).wait()
    return carry

  lax.fori_loop(0, (HIST - 1) * G, drain, 0)

  pltpu.sync_copy(acc_v, out_hbm.at[pl.ds(g0, G)])


_pool_call = pl.kernel(
    _pool_body,
    out_type=jax.ShapeDtypeStruct((NW * G, S, D), jnp.float32),
    mesh=plsc.VectorSubcoreMesh(core_axis_name="c", subcore_axis_name="s"),
    scratch_types=[
        pltpu.VMEM((HIST, G, S), jnp.int32),
        pltpu.VMEM((G, S, D), jnp.float32),
        pltpu.SemaphoreType.DMA,
        pltpu.SemaphoreType.DMA,
    ],
    compiler_params=pltpu.CompilerParams(use_tc_tiling_on_sc=False),
)


_V = 1000000
_W = 32768
_H = _W // 2
_NB = (_V + _W - 1) // _W


def _detile_body(a_ref, o_ref):
  x = a_ref[...]
  o_ref[...] = jnp.concatenate([x[:, :_H], x[:, _H:]], axis=0).T


def _detile(table_t):
  return pl.pallas_call(
      _detile_body,
      grid=(_NB,),
      in_specs=[pl.BlockSpec((64, _W), lambda i: (0, i))],
      out_specs=pl.BlockSpec((_H, 128), lambda i: (i, 0)),
      out_shape=jax.ShapeDtypeStruct((_NB * _H, 128), jnp.float32),
  )(table_t)


def _mlp_body(x_ref, w1t_ref, b1_ref, w2_ref, b2_ref, o_ref):
  x = x_ref[...] * (1.0 / HIST)
  h = jnp.dot(x, w1t_ref[...], preferred_element_type=jnp.float32)
  h = jnp.maximum(h + b1_ref[...], 0.0)
  z = jnp.sum(h * w2_ref[...], axis=1, keepdims=True) + b2_ref[0, 0]
  o_ref[...] = 1.0 / (1.0 + jnp.exp(-z))


@jax.jit
def kernel(tag_idxs, table, W1, b1, W2, b2):
  batch = tag_idxs.shape[0]
  lin = _detile(table.T).reshape(2 * _NB * _H, D)
  t = tag_idxs.astype(jnp.int32)
  r = t & (_W - 1)
  t = (t - r) + ((r & (_H - 1)) << 1) + (r >> (_H.bit_length() - 1))
  idx3 = t.T.reshape(HIST, NW * G, S)
  pooled = _pool_call(idx3, lin).reshape(batch, D)
  out = pl.pallas_call(
      _mlp_body,
      out_shape=jax.ShapeDtypeStruct((batch, 1), jnp.float32),
  )(pooled, W1.T, b1.reshape(1, D), W2, b2.reshape(1, 1))
  return out
